# baseline (speedup 1.0000x reference)
import functools
import math

import jax
import jax.numpy as jnp
from jax import lax
from jax.experimental import pallas as pl
from jax.experimental.pallas import tpu as pltpu
from jax.experimental.pallas import tpu_sc as plsc

N_MODES = 64
DIM = 64
NC = 2
NS = 16
NW = NC * NS
L = 16
CH = 256


def _sc_kernel(n):
    mesh = plsc.VectorSubcoreMesh(core_axis_name="c", subcore_axis_name="s")
    per_w = n // NW
    nch = per_w // CH

    @functools.partial(
        pl.kernel,
        mesh=mesh,
        compiler_params=pltpu.CompilerParams(needs_layout_passes=False),
        out_type=[
            jax.ShapeDtypeStruct((DIM, n), jnp.float32),
            jax.ShapeDtypeStruct((n,), jnp.float32),
        ],
        scratch_types=[
            pltpu.VMEM((N_MODES * DIM,), jnp.float32),
            pltpu.VMEM((N_MODES * DIM,), jnp.float32),
            pltpu.VMEM((L,), jnp.float32),
            pltpu.VMEM((CH,), jnp.int32),
            pltpu.VMEM((CH,), jnp.int32),
            pltpu.VMEM((DIM, CH), jnp.float32),
            pltpu.VMEM((DIM, CH), jnp.float32),
            pltpu.VMEM((DIM, CH), jnp.float32),
            pltpu.VMEM((DIM, CH), jnp.float32),
            pltpu.VMEM((CH,), jnp.float32),
            pltpu.VMEM((CH,), jnp.float32),
            pltpu.SemaphoreType.DMA,
            pltpu.SemaphoreType.DMA,
            pltpu.SemaphoreType.DMA,
            pltpu.SemaphoreType.DMA,
            pltpu.SemaphoreType.DMA,
            pltpu.SemaphoreType.DMA,
            pltpu.SemaphoreType.DMA,
            pltpu.SemaphoreType.DMA,
        ],
    )
    def k(ls_hbm, loc_hbm, c_hbm, idx_hbm, eps_hbm,
          z_hbm, lp_hbm,
          scale_v, loc_v, c_v,
          idx0_v, idx1_v, eps0_v, eps1_v, z0_v, z1_v, lp0_v, lp1_v,
          ei0_s, ei1_s, ii0_s, ii1_s, zo0_s, zo1_s, lo0_s, lo1_s):
        wid = lax.axis_index("s") * NC + lax.axis_index("c")
        idx_b = (idx0_v, idx1_v)
        eps_b = (eps0_v, eps1_v)
        z_b = (z0_v, z1_v)
        lp_b = (lp0_v, lp1_v)
        ei_s = (ei0_s, ei1_s)
        ii_s = (ii0_s, ii1_s)
        zo_s = (zo0_s, zo1_s)
        lo_s = (lo0_s, lo1_s)

        pltpu.sync_copy(ls_hbm, scale_v)
        pltpu.sync_copy(loc_hbm, loc_v)
        pltpu.sync_copy(c_hbm, c_v)

        def exp_body(i, _):
            scale_v[pl.ds(i * L, L)] = jnp.exp(scale_v[pl.ds(i * L, L)])
            return 0
        lax.fori_loop(0, (N_MODES * DIM) // L, exp_body, 0)

        cvec = c_v[...]
        wbase = wid * per_w

        def in_copies(ci, b):
            sbase = wbase + ci * CH
            return (
                pltpu.make_async_copy(
                    eps_hbm.at[:, pl.ds(sbase, CH)], eps_b[b], ei_s[b]),
                pltpu.make_async_copy(
                    idx_hbm.at[pl.ds(sbase, CH)], idx_b[b], ii_s[b]),
            )

        def out_copies(ci, b):
            sbase = wbase + ci * CH
            return (
                pltpu.make_async_copy(
                    z_b[b], z_hbm.at[:, pl.ds(sbase, CH)], zo_s[b]),
                pltpu.make_async_copy(
                    lp_b[b], lp_hbm.at[pl.ds(sbase, CH)], lo_s[b]),
            )

        def start(copies):
            for c in copies:
                c.start()

        def wait(copies):
            for c in copies:
                c.wait()

        def compute(b):
            idx_v, eps_v, z_v, lp_v = idx_b[b], eps_b[b], z_b[b], lp_b[b]

            @plsc.parallel_loop(0, CH // L, unroll=3)
            def group_body(g):
                mvec = idx_v[pl.ds(g * L, L)]
                acc0 = jnp.zeros((L,), jnp.float32)
                acc1 = jnp.zeros((L,), jnp.float32)
                for d in range(DIM):
                    tidx = mvec + (d * N_MODES)
                    sv = plsc.load_gather(scale_v, [tidx])
                    lv = plsc.load_gather(loc_v, [tidx])
                    ev = eps_v[d, pl.ds(g * L, L)]
                    z_v[d, pl.ds(g * L, L)] = ev * sv + lv
                    if d % 2 == 0:
                        acc0 = acc0 + ev * ev
                    else:
                        acc1 = acc1 + ev * ev
                lp_v[pl.ds(g * L, L)] = cvec - 0.5 * (acc0 + acc1)

        start(in_copies(0, 0))

        def pair_body(p, _):
            ci0 = p * 2
            ci1 = ci0 + 1
            start(in_copies(ci1, 1))
            wait(in_copies(ci0, 0))

            @pl.when(p > 0)
            def _():
                wait(out_copies(ci0 - 2, 0))
            compute(0)
            start(out_copies(ci0, 0))

            @pl.when(p < nch // 2 - 1)
            def _():
                start(in_copies(ci0 + 2, 0))
            wait(in_copies(ci1, 1))

            @pl.when(p > 0)
            def _():
                wait(out_copies(ci1 - 2, 1))
            compute(1)
            start(out_copies(ci1, 1))
            return 0
        lax.fori_loop(0, nch // 2, pair_body, 0)
        wait(out_copies(nch - 2, 0))
        wait(out_copies(nch - 1, 1))

    return k


def kernel(eps, loc, log_scale, weight_scores, mode_ind, num_samples):
    n = eps.shape[0]
    log_w = jax.nn.log_softmax(weight_scores, axis=1)
    per_mode = log_w - jnp.sum(log_scale, axis=2)
    c = (-0.5 * DIM * math.log(2.0 * math.pi)
         + jax.scipy.special.logsumexp(per_mode, axis=1))
    c_arr = jnp.broadcast_to(c.astype(jnp.float32), (L,))

    ls_cm = jnp.swapaxes(log_scale[0], 0, 1).reshape(-1)
    loc_cm = jnp.swapaxes(loc[0], 0, 1).reshape(-1)

    z_t, log_p = _sc_kernel(n)(
        ls_cm.astype(jnp.float32),
        loc_cm.astype(jnp.float32),
        c_arr,
        mode_ind.astype(jnp.int32),
        eps.T,
    )
    return z_t.T, log_p


# speedup vs baseline: 1.2767x; 1.2767x over previous
import functools
import math

import jax
import jax.numpy as jnp
from jax import lax
from jax.experimental import pallas as pl
from jax.experimental.pallas import tpu as pltpu
from jax.experimental.pallas import tpu_sc as plsc

N_MODES = 64
DIM = 64
NC = 2
NS = 16
NW = NC * NS
L = 16
CH = 256


def _sc_kernel(n):
    mesh = plsc.VectorSubcoreMesh(core_axis_name="c", subcore_axis_name="s")
    per_w = n // NW
    nch = per_w // CH

    @functools.partial(
        pl.kernel,
        mesh=mesh,
        compiler_params=pltpu.CompilerParams(needs_layout_passes=False),
        out_type=[
            jax.ShapeDtypeStruct((DIM, n), jnp.float32),
            jax.ShapeDtypeStruct((n,), jnp.float32),
        ],
        scratch_types=[
            pltpu.VMEM((N_MODES * DIM,), jnp.float32),
            pltpu.VMEM((N_MODES * DIM,), jnp.float32),
            pltpu.VMEM((L,), jnp.float32),
            pltpu.VMEM((CH,), jnp.int32),
            pltpu.VMEM((CH,), jnp.int32),
            pltpu.VMEM((DIM, CH), jnp.float32),
            pltpu.VMEM((DIM, CH), jnp.float32),
            pltpu.VMEM((DIM, CH), jnp.float32),
            pltpu.VMEM((DIM, CH), jnp.float32),
            pltpu.VMEM((CH,), jnp.float32),
            pltpu.VMEM((CH,), jnp.float32),
            pltpu.SemaphoreType.DMA,
            pltpu.SemaphoreType.DMA,
            pltpu.SemaphoreType.DMA,
            pltpu.SemaphoreType.DMA,
            pltpu.SemaphoreType.DMA,
            pltpu.SemaphoreType.DMA,
            pltpu.SemaphoreType.DMA,
            pltpu.SemaphoreType.DMA,
        ],
    )
    def k(ls_hbm, loc_hbm, c_hbm, idx_hbm, eps_hbm,
          z_hbm, lp_hbm,
          scale_v, loc_v, c_v,
          idx0_v, idx1_v, eps0_v, eps1_v, z0_v, z1_v, lp0_v, lp1_v,
          ei0_s, ei1_s, ii0_s, ii1_s, zo0_s, zo1_s, lo0_s, lo1_s):
        wid = lax.axis_index("s") * NC + lax.axis_index("c")
        idx_b = (idx0_v, idx1_v)
        eps_b = (eps0_v, eps1_v)
        z_b = (z0_v, z1_v)
        lp_b = (lp0_v, lp1_v)
        ei_s = (ei0_s, ei1_s)
        ii_s = (ii0_s, ii1_s)
        zo_s = (zo0_s, zo1_s)
        lo_s = (lo0_s, lo1_s)

        pltpu.sync_copy(ls_hbm, scale_v)
        pltpu.sync_copy(loc_hbm, loc_v)
        pltpu.sync_copy(c_hbm, c_v)

        def exp_body(i, _):
            scale_v[pl.ds(i * L, L)] = jnp.exp(scale_v[pl.ds(i * L, L)])
            return 0
        lax.fori_loop(0, (N_MODES * DIM) // L, exp_body, 0)

        cvec = c_v[...]
        wbase = wid * per_w

        def in_copies(ci, b):
            sbase = wbase + ci * CH
            return (
                pltpu.make_async_copy(
                    eps_hbm.at[:, pl.ds(sbase, CH)], eps_b[b], ei_s[b]),
                pltpu.make_async_copy(
                    idx_hbm.at[pl.ds(sbase, CH)], idx_b[b], ii_s[b]),
            )

        def out_copies(ci, b):
            sbase = wbase + ci * CH
            return (
                pltpu.make_async_copy(
                    z_b[b], z_hbm.at[:, pl.ds(sbase, CH)], zo_s[b]),
                pltpu.make_async_copy(
                    lp_b[b], lp_hbm.at[pl.ds(sbase, CH)], lo_s[b]),
            )

        def start(copies):
            for c in copies:
                c.start()

        def wait(copies):
            for c in copies:
                c.wait()

        def compute(b):
            idx_v, eps_v, z_v, lp_v = idx_b[b], eps_b[b], z_b[b], lp_b[b]

            @plsc.parallel_loop(0, CH // L, unroll=2)
            def group_body(g):
                mvec = idx_v[pl.ds(g * L, L)]
                acc0 = jnp.zeros((L,), jnp.float32)
                acc1 = jnp.zeros((L,), jnp.float32)
                for d in range(DIM):
                    tidx = mvec + (d * N_MODES)
                    sv = plsc.load_gather(scale_v, [tidx])
                    lv = plsc.load_gather(loc_v, [tidx])
                    ev = eps_v[d, pl.ds(g * L, L)]
                    z_v[d, pl.ds(g * L, L)] = ev * sv + lv
                    if d % 2 == 0:
                        acc0 = acc0 + ev * ev
                    else:
                        acc1 = acc1 + ev * ev
                lp_v[pl.ds(g * L, L)] = cvec - 0.5 * (acc0 + acc1)

        start(in_copies(0, 0))

        def pair_body(p, _):
            ci0 = p * 2
            ci1 = ci0 + 1
            start(in_copies(ci1, 1))
            wait(in_copies(ci0, 0))

            @pl.when(p > 0)
            def _():
                wait(out_copies(ci0 - 2, 0))
            compute(0)
            start(out_copies(ci0, 0))

            @pl.when(p < nch // 2 - 1)
            def _():
                start(in_copies(ci0 + 2, 0))
            wait(in_copies(ci1, 1))

            @pl.when(p > 0)
            def _():
                wait(out_copies(ci1 - 2, 1))
            compute(1)
            start(out_copies(ci1, 1))
            return 0
        lax.fori_loop(0, nch // 2, pair_body, 0)
        wait(out_copies(nch - 2, 0))
        wait(out_copies(nch - 1, 1))

    return k


def kernel(eps, loc, log_scale, weight_scores, mode_ind, num_samples):
    n = eps.shape[0]
    log_w = jax.nn.log_softmax(weight_scores, axis=1)
    per_mode = log_w - jnp.sum(log_scale, axis=2)
    c = (-0.5 * DIM * math.log(2.0 * math.pi)
         + jax.scipy.special.logsumexp(per_mode, axis=1))
    c_arr = jnp.broadcast_to(c.astype(jnp.float32), (L,))

    ls_cm = jnp.swapaxes(log_scale[0], 0, 1).reshape(-1)
    loc_cm = jnp.swapaxes(loc[0], 0, 1).reshape(-1)

    z_t, log_p = _sc_kernel(n)(
        ls_cm.astype(jnp.float32),
        loc_cm.astype(jnp.float32),
        c_arr,
        mode_ind.astype(jnp.int32),
        eps.T,
    )
    return z_t.T, log_p
